# baseline (speedup 1.0000x reference)
import functools

import jax
import jax.numpy as jnp
from jax import lax
from jax.experimental import pallas as pl
from jax.experimental.pallas import tpu as pltpu
from jax.experimental.pallas import tpu_sc as plsc

B, C, H, W = 8, 256, 32, 32
K = 8192
BETA = 0.25

N = B * H * W
BLK_N = 256
NB = N // BLK_N
NWIN = 4
WSZ = K // NWIN


def _argmin_body(x_ref, cb_ref, rn_ref, cn_ref, idx_ref, loss_ref):
    i = pl.program_id(0)
    mm = lax.dot_general(
        x_ref[...], cb_ref[...],
        dimension_numbers=(((1,), (1,)), ((), ())),
        preferred_element_type=jnp.float32,
    )
    d = (rn_ref[...] + cn_ref[...]) - mm

    ms, as_ = [], []
    for s in range(NWIN):
        dw = d[:, s * WSZ:(s + 1) * WSZ]
        mw = jnp.min(dw, axis=1, keepdims=True)
        iota = lax.broadcasted_iota(jnp.int32, (BLK_N, WSZ), 1) + s * WSZ
        aw = jnp.min(jnp.where(dw == mw, iota, K), axis=1)
        ms.append(mw[:, 0])
        as_.append(aw)

    M = ms[0].astype(jnp.bfloat16).astype(jnp.float32)
    I = as_[0]
    V = ms[0]
    for s in range(1, NWIN):
        pick = (ms[s] < M) | ((ms[s] == M) & (as_[s] < I))
        I = jnp.where(pick, as_[s], I)
        V = jnp.where(pick, ms[s], V)
        M = jnp.where(pick, ms[s].astype(jnp.bfloat16).astype(jnp.float32), M)

    idx_ref[0, 0, :] = I

    @pl.when(i == 0)
    def _():
        loss_ref[...] = jnp.zeros((1, 1), jnp.float32)

    loss_ref[...] += jnp.sum(V).reshape(1, 1)


def _search(lhs_bf, rhs_bf, rn, cn):
    return pl.pallas_call(
        _argmin_body,
        grid=(NB,),
        in_specs=[
            pl.BlockSpec((BLK_N, C), lambda i: (i, 0)),
            pl.BlockSpec((K, C), lambda i: (0, 0)),
            pl.BlockSpec((BLK_N, 1), lambda i: (i, 0)),
            pl.BlockSpec((1, K), lambda i: (0, 0)),
        ],
        out_specs=[
            pl.BlockSpec((1, 1, BLK_N), lambda i: (i, 0, 0)),
            pl.BlockSpec((1, 1), lambda i: (0, 0)),
        ],
        out_shape=[
            jax.ShapeDtypeStruct((NB, 1, BLK_N), jnp.int32),
            jax.ShapeDtypeStruct((1, 1), jnp.float32),
        ],
        compiler_params=pltpu.CompilerParams(
            dimension_semantics=("arbitrary",),
        ),
    )(lhs_bf, rhs_bf, rn, cn)


_NW = 32
_BPW = N // _NW
_SC_MESH = plsc.VectorSubcoreMesh(core_axis_name="c", subcore_axis_name="s")


@functools.partial(
    pl.kernel,
    mesh=_SC_MESH,
    out_type=jax.ShapeDtypeStruct((N, C), jnp.float32),
    scratch_types=[
        pltpu.VMEM((2, _BPW // 2), jnp.int32),
        pltpu.VMEM((_BPW, C), jnp.float32),
        pltpu.SemaphoreType.DMA,
    ],
)
def _sc_gather(table_hbm, idx_hbm, out_hbm, idx_v, rows_v, sem):
    wid = lax.axis_index("s") * 2 + lax.axis_index("c")
    pltpu.sync_copy(idx_hbm.at[pl.ds(2 * wid, 2)], idx_v)
    half = _BPW // 2
    for j in range(2):
        pltpu.async_copy(
            table_hbm.at[idx_v.at[j]], rows_v.at[pl.ds(j * half, half)], sem
        ).wait()
    pltpu.sync_copy(rows_v, out_hbm.at[pl.ds(wid * _BPW, _BPW)])


def kernel(x, codebook):
    b, c, h, w = x.shape
    xt = jnp.transpose(x, (0, 2, 3, 1)).reshape(b, h * w, c)
    cb = codebook[:-1]
    flat = xt.reshape(-1, c)
    rn = jnp.sum(flat ** 2, axis=1, keepdims=True)
    cn = jnp.sum(cb ** 2, axis=1).reshape(1, K)
    lhs_bf = (2.0 * flat).astype(jnp.bfloat16)
    rhs_bf = cb.astype(jnp.bfloat16)

    idx3, loss_sum = _search(lhs_bf, rhs_bf, rn, cn)
    idx = idx3.reshape(-1)

    x_q = _sc_gather(codebook, idx.reshape(2 * _NW, _BPW // 2))
    lm = loss_sum[0, 0] / (N * C)
    loss = BETA * lm + lm
    x_q_st = flat + (x_q - flat)
    x_q_out = jnp.transpose(x_q_st.reshape(b, h, w, c), (0, 3, 1, 2))
    code = idx.reshape(b, h, w)
    return (x_q_out, loss, code)


# speedup vs baseline: 1.0397x; 1.0397x over previous
import functools

import jax
import jax.numpy as jnp
from jax import lax
from jax.experimental import pallas as pl
from jax.experimental.pallas import tpu as pltpu
from jax.experimental.pallas import tpu_sc as plsc

B, C, H, W = 8, 256, 32, 32
K = 8192
BETA = 0.25

N = B * H * W
BLK_N = 256
NB = N // BLK_N
NWIN = 4
WSZ = K // NWIN


def _argmin_body(x_ref, cb_ref, rn_ref, cn_ref, idx_ref, loss_ref):
    i = pl.program_id(0)
    xb = (2.0 * x_ref[...]).astype(jnp.bfloat16)
    mm = lax.dot_general(
        xb, cb_ref[...],
        dimension_numbers=(((1,), (1,)), ((), ())),
        preferred_element_type=jnp.float32,
    )
    d = (rn_ref[...] + cn_ref[...]) - mm

    ms, as_ = [], []
    for s in range(NWIN):
        dw = d[:, s * WSZ:(s + 1) * WSZ]
        mw = jnp.min(dw, axis=1, keepdims=True)
        iota = lax.broadcasted_iota(jnp.int32, (BLK_N, WSZ), 1) + s * WSZ
        aw = jnp.min(jnp.where(dw == mw, iota, K), axis=1)
        ms.append(mw[:, 0])
        as_.append(aw)

    M = ms[0].astype(jnp.bfloat16).astype(jnp.float32)
    I = as_[0]
    V = ms[0]
    for s in range(1, NWIN):
        pick = ms[s] < M
        I = jnp.where(pick, as_[s], I)
        V = jnp.where(pick, ms[s], V)
        M = jnp.where(pick, ms[s].astype(jnp.bfloat16).astype(jnp.float32), M)

    idx_ref[0, 0, :] = I

    @pl.when(i == 0)
    def _():
        loss_ref[...] = jnp.zeros((1, 1), jnp.float32)

    loss_ref[...] += jnp.sum(V).reshape(1, 1)


def _search(lhs_bf, rhs_bf, rn, cn):
    return pl.pallas_call(
        _argmin_body,
        grid=(NB,),
        in_specs=[
            pl.BlockSpec((BLK_N, C), lambda i: (i, 0)),
            pl.BlockSpec((K, C), lambda i: (0, 0)),
            pl.BlockSpec((BLK_N, 1), lambda i: (i, 0)),
            pl.BlockSpec((1, K), lambda i: (0, 0)),
        ],
        out_specs=[
            pl.BlockSpec((1, 1, BLK_N), lambda i: (i, 0, 0)),
            pl.BlockSpec((1, 1), lambda i: (0, 0)),
        ],
        out_shape=[
            jax.ShapeDtypeStruct((NB, 1, BLK_N), jnp.int32),
            jax.ShapeDtypeStruct((1, 1), jnp.float32),
        ],
        compiler_params=pltpu.CompilerParams(
            dimension_semantics=("arbitrary",),
        ),
    )(lhs_bf, rhs_bf, rn, cn)


_NW = 32
_BPW = N // _NW
_SC_MESH = plsc.VectorSubcoreMesh(core_axis_name="c", subcore_axis_name="s")


@functools.partial(
    pl.kernel,
    mesh=_SC_MESH,
    out_type=jax.ShapeDtypeStruct((N, C), jnp.float32),
    scratch_types=[
        pltpu.VMEM((2, _BPW // 2), jnp.int32),
        pltpu.VMEM((_BPW, C), jnp.float32),
        pltpu.SemaphoreType.DMA,
    ],
)
def _sc_gather(table_hbm, idx_hbm, out_hbm, idx_v, rows_v, sem):
    wid = lax.axis_index("s") * 2 + lax.axis_index("c")
    pltpu.sync_copy(idx_hbm.at[pl.ds(2 * wid, 2)], idx_v)
    half = _BPW // 2
    for j in range(2):
        pltpu.async_copy(
            table_hbm.at[idx_v.at[j]], rows_v.at[pl.ds(j * half, half)], sem
        ).wait()
    pltpu.sync_copy(rows_v, out_hbm.at[pl.ds(wid * _BPW, _BPW)])


def kernel(x, codebook):
    b, c, h, w = x.shape
    xt = jnp.transpose(x, (0, 2, 3, 1)).reshape(b, h * w, c)
    cb = codebook[:-1]
    flat = xt.reshape(-1, c)
    rn = jnp.sum(flat ** 2, axis=1, keepdims=True)
    cn = jnp.sum(cb ** 2, axis=1).reshape(1, K)
    rhs_bf = cb.astype(jnp.bfloat16)

    idx3, loss_sum = _search(flat, rhs_bf, rn, cn)
    idx = idx3.reshape(-1)

    x_q = _sc_gather(codebook, idx.reshape(2 * _NW, _BPW // 2))
    lm = loss_sum[0, 0] / (N * C)
    loss = BETA * lm + lm
    x_q_st = flat + (x_q - flat)
    x_q_out = jnp.transpose(x_q_st.reshape(b, h, w, c), (0, 3, 1, 2))
    code = idx.reshape(b, h, w)
    return (x_q_out, loss, code)


# speedup vs baseline: 1.0408x; 1.0011x over previous
import functools

import jax
import jax.numpy as jnp
from jax import lax
from jax.experimental import pallas as pl
from jax.experimental.pallas import tpu as pltpu
from jax.experimental.pallas import tpu_sc as plsc

B, C, H, W = 8, 256, 32, 32
K = 8192
BETA = 0.25

N = B * H * W
BLK_N = 256
NB = N // BLK_N
NWIN = 4
WSZ = K // NWIN


def _argmin_body(x_ref, cb_ref, rn_ref, cn_ref, idx_ref, loss_ref):
    i = pl.program_id(0)
    xb = (2.0 * x_ref[...]).astype(jnp.bfloat16)
    mm = lax.dot_general(
        xb, cb_ref[...],
        dimension_numbers=(((1,), (1,)), ((), ())),
        preferred_element_type=jnp.float32,
    )
    d = (rn_ref[...] + cn_ref[...]) - mm

    ms, as_ = [], []
    for s in range(NWIN):
        dw = d[:, s * WSZ:(s + 1) * WSZ]
        mw = jnp.min(dw, axis=1, keepdims=True)
        iota = lax.broadcasted_iota(jnp.int32, (BLK_N, WSZ), 1) + s * WSZ
        aw = jnp.min(jnp.where(dw == mw, iota, K), axis=1)
        ms.append(mw[:, 0])
        as_.append(aw)

    M = ms[0].astype(jnp.bfloat16).astype(jnp.float32)
    I = as_[0]
    V = ms[0]
    for s in range(1, NWIN):
        pick = ms[s] < M
        I = jnp.where(pick, as_[s], I)
        V = jnp.where(pick, ms[s], V)
        M = jnp.where(pick, ms[s].astype(jnp.bfloat16).astype(jnp.float32), M)

    idx_ref[0, 0, :] = I

    @pl.when(i == 0)
    def _():
        loss_ref[...] = jnp.zeros((1, 1), jnp.float32)

    loss_ref[...] += jnp.sum(V).reshape(1, 1)


def _search(lhs_bf, rhs_bf, rn, cn):
    return pl.pallas_call(
        _argmin_body,
        grid=(NB,),
        in_specs=[
            pl.BlockSpec((BLK_N, C), lambda i: (i, 0)),
            pl.BlockSpec((K, C), lambda i: (0, 0)),
            pl.BlockSpec((BLK_N, 1), lambda i: (i, 0)),
            pl.BlockSpec((1, K), lambda i: (0, 0)),
        ],
        out_specs=[
            pl.BlockSpec((1, 1, BLK_N), lambda i: (i, 0, 0)),
            pl.BlockSpec((1, 1), lambda i: (0, 0)),
        ],
        out_shape=[
            jax.ShapeDtypeStruct((NB, 1, BLK_N), jnp.int32),
            jax.ShapeDtypeStruct((1, 1), jnp.float32),
        ],
        compiler_params=pltpu.CompilerParams(
            dimension_semantics=("arbitrary",),
        ),
    )(lhs_bf, rhs_bf, rn, cn)


_NW = 32
_BPW = N // _NW
_SC_MESH = plsc.VectorSubcoreMesh(core_axis_name="c", subcore_axis_name="s")


@functools.partial(
    pl.kernel,
    mesh=_SC_MESH,
    out_type=jax.ShapeDtypeStruct((N, C), jnp.float32),
    scratch_types=[
        pltpu.VMEM((2, _BPW // 2), jnp.int32),
        pltpu.VMEM((_BPW, C), jnp.float32),
        pltpu.SemaphoreType.DMA,
    ],
)
def _sc_gather(table_hbm, idx_hbm, out_hbm, idx_v, rows_v, sem):
    wid = lax.axis_index("s") * 2 + lax.axis_index("c")
    pltpu.sync_copy(idx_hbm.at[pl.ds(2 * wid, 2)], idx_v)
    half = _BPW // 2
    copies = [
        pltpu.async_copy(
            table_hbm.at[idx_v.at[j]], rows_v.at[pl.ds(j * half, half)], sem
        )
        for j in range(2)
    ]
    for cp in copies:
        cp.wait()
    pltpu.sync_copy(rows_v, out_hbm.at[pl.ds(wid * _BPW, _BPW)])


def kernel(x, codebook):
    b, c, h, w = x.shape
    xt = jnp.transpose(x, (0, 2, 3, 1)).reshape(b, h * w, c)
    cb = codebook[:-1]
    flat = xt.reshape(-1, c)
    rn = jnp.sum(flat ** 2, axis=1, keepdims=True)
    cn = jnp.sum(cb ** 2, axis=1).reshape(1, K)
    rhs_bf = cb.astype(jnp.bfloat16)

    idx3, loss_sum = _search(flat, rhs_bf, rn, cn)
    idx = idx3.reshape(-1)

    x_q = _sc_gather(codebook, idx.reshape(2 * _NW, _BPW // 2))
    lm = loss_sum[0, 0] / (N * C)
    loss = BETA * lm + lm
    x_q_st = flat + (x_q - flat)
    x_q_out = jnp.transpose(x_q_st.reshape(b, h, w, c), (0, 3, 1, 2))
    code = idx.reshape(b, h, w)
    return (x_q_out, loss, code)


# speedup vs baseline: 1.1068x; 1.0634x over previous
import functools

import jax
import jax.numpy as jnp
from jax import lax
from jax.experimental import pallas as pl
from jax.experimental.pallas import tpu as pltpu
from jax.experimental.pallas import tpu_sc as plsc

B, C, H, W = 8, 256, 32, 32
K = 8192
BETA = 0.25

N = B * H * W
BLK_N = 256
NB = N // BLK_N
NWIN = 4
WSZ = K // NWIN


def _argmin_body(x_ref, cb_ref, rn_ref, cn_ref, idx_ref, loss_ref):
    i = pl.program_id(0)
    xb = (2.0 * x_ref[...]).astype(jnp.bfloat16)
    mm = lax.dot_general(
        xb, cb_ref[...],
        dimension_numbers=(((1,), (1,)), ((), ())),
        preferred_element_type=jnp.float32,
    )
    d = (rn_ref[...] + cn_ref[...]) - mm

    ms, as_ = [], []
    for s in range(NWIN):
        dw = d[:, s * WSZ:(s + 1) * WSZ]
        mw = jnp.min(dw, axis=1, keepdims=True)
        iota = lax.broadcasted_iota(jnp.int32, (BLK_N, WSZ), 1) + s * WSZ
        aw = jnp.min(jnp.where(dw == mw, iota, K), axis=1)
        ms.append(mw[:, 0])
        as_.append(aw)

    M = ms[0].astype(jnp.bfloat16).astype(jnp.float32)
    I = as_[0]
    V = ms[0]
    for s in range(1, NWIN):
        pick = ms[s] < M
        I = jnp.where(pick, as_[s], I)
        V = jnp.where(pick, ms[s], V)
        M = jnp.where(pick, ms[s].astype(jnp.bfloat16).astype(jnp.float32), M)

    idx_ref[0, 0, :] = I

    @pl.when(i == 0)
    def _():
        loss_ref[...] = jnp.zeros((1, 1), jnp.float32)

    loss_ref[...] += jnp.sum(V).reshape(1, 1)


def _search(lhs_bf, rhs_bf, rn, cn):
    return pl.pallas_call(
        _argmin_body,
        grid=(NB,),
        in_specs=[
            pl.BlockSpec((BLK_N, C), lambda i: (i, 0)),
            pl.BlockSpec((K, C), lambda i: (0, 0)),
            pl.BlockSpec((BLK_N, 1), lambda i: (i, 0)),
            pl.BlockSpec((1, K), lambda i: (0, 0)),
        ],
        out_specs=[
            pl.BlockSpec((1, 1, BLK_N), lambda i: (i, 0, 0)),
            pl.BlockSpec((1, 1), lambda i: (0, 0)),
        ],
        out_shape=[
            jax.ShapeDtypeStruct((NB, 1, BLK_N), jnp.int32),
            jax.ShapeDtypeStruct((1, 1), jnp.float32),
        ],
        compiler_params=pltpu.CompilerParams(
            dimension_semantics=("arbitrary",),
        ),
    )(lhs_bf, rhs_bf, rn, cn)


_NW = 32
_BPW = N // _NW
_SC_MESH = plsc.VectorSubcoreMesh(core_axis_name="c", subcore_axis_name="s")


@functools.partial(
    pl.kernel,
    mesh=_SC_MESH,
    out_type=jax.ShapeDtypeStruct((N, C), jnp.float32),
    scratch_types=[
        pltpu.VMEM((2, _BPW // 2), jnp.int32),
        pltpu.VMEM((_BPW, C), jnp.float32),
        pltpu.SemaphoreType.DMA,
    ],
)
def _sc_gather(table_hbm, idx_hbm, out_hbm, idx_v, rows_v, sem):
    wid = lax.axis_index("s") * 2 + lax.axis_index("c")
    pltpu.sync_copy(idx_hbm.at[pl.ds(2 * wid, 2)], idx_v)
    half = _BPW // 2
    copies = [
        pltpu.async_copy(
            table_hbm.at[idx_v.at[j]], rows_v.at[pl.ds(j * half, half)], sem
        )
        for j in range(2)
    ]
    for cp in copies:
        cp.wait()
    pltpu.sync_copy(rows_v, out_hbm.at[pl.ds(wid * _BPW, _BPW)])


def kernel(x, codebook):
    b, c, h, w = x.shape
    xt = jnp.transpose(x, (0, 2, 3, 1)).reshape(b, h * w, c)
    cb = codebook[:-1]
    flat = xt.reshape(-1, c)
    rn = jnp.sum(flat ** 2, axis=1, keepdims=True)
    cn = jnp.sum(cb ** 2, axis=1).reshape(1, K)
    rhs_bf = cb.astype(jnp.bfloat16)

    idx3, loss_sum = _search(flat, rhs_bf, rn, cn)
    idx = idx3.reshape(-1)

    x_q = _sc_gather(codebook, idx.reshape(2 * _NW, _BPW // 2))
    lm = loss_sum[0, 0] / (N * C)
    loss = BETA * lm + lm
    x_q_out = jnp.transpose(x_q.reshape(b, h, w, c), (0, 3, 1, 2))
    code = idx.reshape(b, h, w)
    return (x_q_out, loss, code)
